# baseline (speedup 1.0000x reference)
import jax
import jax.numpy as jnp
from jax.experimental import pallas as pl
from jax.experimental.pallas import tpu as pltpu

_BM = 400
_NBUF = 3
_NOBUF = 2


def _gconv_stream_kernel(flt_hbm, x_ref, w_ref, o_hbm, buf, obuf, sems, osems):
    n_rows = flt_hbm.shape[0]
    nchunks = n_rows // _BM

    def copy_in(c, slot):
        return pltpu.make_async_copy(
            flt_hbm.at[pl.ds(c * _BM, _BM), :], buf.at[slot], sems.at[slot])

    def copy_out(c, slot):
        return pltpu.make_async_copy(
            obuf.at[slot], o_hbm.at[pl.ds(c * _BM, _BM), :], osems.at[slot])

    for s in range(min(_NBUF - 1, nchunks)):
        copy_in(s, s).start()

    for c in range(nchunks):
        slot = c % _NBUF
        nxt = c + _NBUF - 1
        if nxt < nchunks:
            copy_in(nxt, nxt % _NBUF).start()
        copy_in(c, slot).wait()
        acc = jnp.dot(buf[slot], x_ref[...],
                      preferred_element_type=jnp.float32)
        lin = jax.lax.dot_general(
            acc, w_ref[...], (((1,), (1,)), ((), ())),
            preferred_element_type=jnp.float32)
        oslot = c % _NOBUF
        if c >= _NOBUF:
            copy_out(c - _NOBUF, oslot).wait()
        obuf[oslot] = jnp.tanh(lin)
        copy_out(c, oslot).start()
    for c in range(max(0, nchunks - _NOBUF), nchunks):
        copy_out(c, c % _NOBUF).wait()


def kernel(inputs, flt, W):
    n_rows, n_cols = flt.shape
    d_in = inputs.shape[1]
    d_out = W.shape[0]
    return pl.pallas_call(
        _gconv_stream_kernel,
        in_specs=[
            pl.BlockSpec(memory_space=pl.ANY),
            pl.BlockSpec((n_cols, d_in), lambda: (0, 0)),
            pl.BlockSpec((d_out, d_in), lambda: (0, 0)),
        ],
        out_specs=pl.BlockSpec(memory_space=pl.ANY),
        out_shape=jax.ShapeDtypeStruct((n_rows, d_out), jnp.float32),
        scratch_shapes=[
            pltpu.VMEM((_NBUF, _BM, n_cols), jnp.float32),
            pltpu.VMEM((_NOBUF, _BM, d_out), jnp.float32),
            pltpu.SemaphoreType.DMA((_NBUF,)),
            pltpu.SemaphoreType.DMA((_NOBUF,)),
        ],
    )(flt, inputs, W)


# speedup vs baseline: 1.0449x; 1.0449x over previous
import jax
import jax.numpy as jnp
from jax.experimental import pallas as pl
from jax.experimental.pallas import tpu as pltpu


def _gconv_block_kernel(flt_ref, x_ref, w_ref, o_ref):
    acc = jnp.dot(flt_ref[...], x_ref[...], preferred_element_type=jnp.float32)
    lin = jax.lax.dot_general(
        acc, w_ref[...], (((1,), (1,)), ((), ())),
        preferred_element_type=jnp.float32)
    o_ref[...] = jnp.tanh(lin)


def _pick_block_rows(n_rows: int) -> int:
    for bm in (400, 200, 80, 40, 16, 8):
        if n_rows % bm == 0:
            return bm
    return n_rows


def kernel(inputs, flt, W):
    n_rows, n_cols = flt.shape
    d_in = inputs.shape[1]
    d_out = W.shape[0]
    bm = _pick_block_rows(n_rows)
    return pl.pallas_call(
        _gconv_block_kernel,
        grid=(n_rows // bm,),
        in_specs=[
            pl.BlockSpec((bm, n_cols), lambda i: (i, 0)),
            pl.BlockSpec((n_cols, d_in), lambda i: (0, 0)),
            pl.BlockSpec((d_out, d_in), lambda i: (0, 0)),
        ],
        out_specs=pl.BlockSpec((bm, d_out), lambda i: (i, 0)),
        out_shape=jax.ShapeDtypeStruct((n_rows, d_out), jnp.float32),
        compiler_params=pltpu.CompilerParams(
            dimension_semantics=("parallel",)),
    )(flt, inputs, W)


# speedup vs baseline: 1.0482x; 1.0032x over previous
import jax
import jax.numpy as jnp
from jax.experimental import pallas as pl
from jax.experimental.pallas import tpu as pltpu


def _gconv_block_kernel(flt_ref, x_ref, w_ref, o_ref):
    acc = jnp.dot(flt_ref[...], x_ref[...], preferred_element_type=jnp.float32)
    lin = jax.lax.dot_general(
        acc, w_ref[...], (((1,), (1,)), ((), ())),
        preferred_element_type=jnp.float32)
    o_ref[...] = jnp.tanh(lin)


def _pick_block_rows(n_rows: int) -> int:
    for bm in (400, 200, 80, 40, 16, 8):
        if n_rows % bm == 0:
            return bm
    return n_rows


def kernel(inputs, flt, W):
    n_rows, n_cols = flt.shape
    d_in = inputs.shape[1]
    d_out = W.shape[0]
    bm = _pick_block_rows(n_rows)
    return pl.pallas_call(
        _gconv_block_kernel,
        grid=(n_rows // bm,),
        in_specs=[
            pl.BlockSpec((bm, n_cols), lambda i: (i, 0)),
            pl.BlockSpec((n_cols, d_in), lambda i: (0, 0)),
            pl.BlockSpec((d_out, d_in), lambda i: (0, 0)),
        ],
        out_specs=pl.BlockSpec((bm, d_out), lambda i: (i, 0)),
        out_shape=jax.ShapeDtypeStruct((n_rows, d_out), jnp.float32),
        compiler_params=pltpu.CompilerParams(
            dimension_semantics=("arbitrary",)),
    )(flt, inputs, W)
